# Initial kernel scaffold; baseline (speedup 1.0000x reference)
import jax
import jax.numpy as jnp
from jax.experimental import pallas as pl


def kernel(source_batch, target_batch, src_x, src_edge_index, src_edge_attr, tgt_x, tgt_edge_index, tgt_edge_attr, W_self, W_nbr, W_edge, b):
    raise NotImplementedError("write your pallas kernel here")


# speedup vs baseline: 1.5915x; 1.5915x over previous
import functools

import jax
import jax.numpy as jnp
from jax import lax
from jax.experimental import pallas as pl
from jax.experimental.pallas import tpu as pltpu
from jax.experimental.pallas import tpu_sc as plsc

NC = 2
NS = 16
NW = NC * NS
BATCH = 128
CHB = 512


def _sc_gather_scatter(x, src, dst, n_pad, k):
    n, d = x.shape
    rows_per_tile = n_pad // NS
    zeros_a = jnp.zeros((rows_per_tile, d), jnp.float32)

    mesh = plsc.VectorSubcoreMesh(
        core_axis_name="c", subcore_axis_name="s", num_cores=NC,
        num_subcores=NS)

    @functools.partial(
        pl.kernel,
        out_type=jax.ShapeDtypeStruct((NC, n_pad, d), jnp.float32),
        mesh=mesh,
        scratch_types=[
            pltpu.VMEM_SHARED((n_pad, d), jnp.float32),
            pltpu.VMEM((2, BATCH), jnp.int32),
            pltpu.VMEM((2, BATCH), jnp.int32),
            pltpu.VMEM((2, BATCH, d), jnp.float32),
            pltpu.SemaphoreType.DMA,
            pltpu.SemaphoreType.DMA,
        ],
    )
    def k_fn(x_hbm, src_hbm, dst_hbm, za_hbm, out_a,
             a_sh, isrc, idst, rows, gsem, ssem):
        cid = lax.axis_index("c")
        sid = lax.axis_index("s")
        wid = cid * NS + sid

        row0 = sid * rows_per_tile
        pltpu.sync_copy(za_hbm, a_sh.at[pl.ds(row0, rows_per_tile)])
        plsc.subcore_barrier()

        def body(g, _):
            base = (wid * k + g) * BATCH
            pltpu.sync_copy(src_hbm.at[pl.ds(base, BATCH)], isrc.at[0])
            pltpu.sync_copy(dst_hbm.at[pl.ds(base, BATCH)], idst.at[0])
            pltpu.async_copy(x_hbm.at[isrc.at[0]], rows.at[0], gsem).wait()
            pltpu.async_copy(rows.at[0], a_sh.at[idst.at[0]], ssem,
                             add=True).wait()
            return _

        lax.fori_loop(0, k, body, None)

        plsc.subcore_barrier()

        pltpu.sync_copy(a_sh.at[pl.ds(row0, rows_per_tile)],
                        out_a.at[cid, pl.ds(row0, rows_per_tile)])

    return k_fn(x, src, dst, zeros_a)


def _sc_edge_attr_sums(dst, ea16, n_pad, k2):
    nb = n_pad * 8 + 8
    zeros_b = jnp.zeros((nb,), jnp.float32)

    mesh = plsc.VectorSubcoreMesh(
        core_axis_name="c", subcore_axis_name="s", num_cores=NC,
        num_subcores=NS)

    @functools.partial(
        pl.kernel,
        out_type=jax.ShapeDtypeStruct((NW, n_pad * 8), jnp.float32),
        mesh=mesh,
        scratch_types=[
            pltpu.VMEM((nb,), jnp.float32),
            pltpu.VMEM((CHB,), jnp.int32),
            pltpu.VMEM((CHB * 16,), jnp.float32),
        ],
    )
    def k_fn(dst_hbm, ea_hbm, zb_hbm, out_b, bacc, dv, eav):
        cid = lax.axis_index("c")
        sid = lax.axis_index("s")
        wid = cid * NS + sid

        pltpu.sync_copy(zb_hbm, bacc)

        def chunk(g, _):
            base = (wid * k2 + g) * CHB
            pltpu.sync_copy(dst_hbm.at[pl.ds(base, CHB)], dv)
            pltpu.sync_copy(ea_hbm.at[pl.ds(base * 16, CHB * 16)], eav)

            def group(q, _):
                dv16 = dv[pl.ds(q * 16, 16)] * 8
                for j in range(16):
                    row = dv16[j]
                    vals = eav[pl.ds((q * 16 + j) * 16, 16)]
                    plsc.addupdate(bacc.at[pl.ds(row, 16)], vals)
                return _

            lax.fori_loop(0, CHB // 16, group, None)
            return _

        lax.fori_loop(0, k2, chunk, None)
        pltpu.sync_copy(bacc.at[pl.ds(0, n_pad * 8)], out_b.at[wid])

    return k_fn(dst, ea16, zeros_b)


def _tc_combine(x, a_part, b_part, w_self, w_nbr, w_edge8, bias):
    n, d = x.shape
    bn = 1000
    grid = (n // bn,)

    def body(x_ref, a_ref, b_ref, ws_ref, wn_ref, we_ref, bias_ref, o_ref):
        agg = a_ref[0] + a_ref[1]
        bsum = jnp.sum(b_ref[...], axis=0)
        wn = wn_ref[...]
        w2 = jnp.dot(we_ref[...], wn, preferred_element_type=jnp.float32)
        acc = jnp.dot(x_ref[...], ws_ref[...],
                      preferred_element_type=jnp.float32)
        acc += jnp.dot(agg, wn, preferred_element_type=jnp.float32)
        acc += jnp.dot(bsum, w2, preferred_element_type=jnp.float32)
        acc += bias_ref[...]
        o_ref[...] = jnp.maximum(acc, 0.0)

    return pl.pallas_call(
        body,
        grid=grid,
        in_specs=[
            pl.BlockSpec((bn, d), lambda i: (i, 0)),
            pl.BlockSpec((NC, bn, d), lambda i: (0, i, 0)),
            pl.BlockSpec((NW, bn, 8), lambda i: (0, i, 0)),
            pl.BlockSpec((d, d), lambda i: (0, 0)),
            pl.BlockSpec((d, d), lambda i: (0, 0)),
            pl.BlockSpec((8, d), lambda i: (0, 0)),
            pl.BlockSpec((1, d), lambda i: (0, 0)),
        ],
        out_specs=pl.BlockSpec((bn, d), lambda i: (i, 0)),
        out_shape=jax.ShapeDtypeStruct((n, d), jnp.float32),
    )(x, a_part, b_part, w_self, w_nbr, w_edge8, bias)


def _gnn(x, edge_index, edge_attr, w_self, w_nbr, w_edge8, bias):
    n, d = x.shape
    e = edge_index.shape[1]
    n_pad = (n + 1 + NS * 8 - 1) // (NS * 8) * (NS * 8)
    step = NW * max(BATCH, CHB)
    e_pad = -(-e // step) * step
    k = e_pad // (NW * BATCH)
    k2 = e_pad // (NW * CHB)

    src = jnp.pad(edge_index[0].astype(jnp.int32), (0, e_pad - e))
    dst = jnp.pad(edge_index[1].astype(jnp.int32), (0, e_pad - e),
                  constant_values=n)
    ea16 = jnp.pad(edge_attr.astype(jnp.float32),
                   ((0, e_pad - e), (0, 16 - edge_attr.shape[1])))

    a_part = _sc_gather_scatter(x, src, dst, n_pad, k)
    b_flat = _sc_edge_attr_sums(dst, ea16.reshape(-1), n_pad, k2)
    b_part = b_flat.reshape(NW, n_pad, 8)
    return _tc_combine(x, a_part, b_part, w_self, w_nbr, w_edge8, bias)


def kernel(source_batch, target_batch, src_x, src_edge_index, src_edge_attr,
           tgt_x, tgt_edge_index, tgt_edge_attr, W_self, W_nbr, W_edge, b):
    w_edge8 = jnp.pad(W_edge.astype(jnp.float32), ((0, 8 - W_edge.shape[0]),
                                                   (0, 0)))
    bias = b.astype(jnp.float32).reshape(1, -1)
    h_src = _gnn(src_x, src_edge_index, src_edge_attr, W_self, W_nbr,
                 w_edge8, bias)
    h_tgt = _gnn(tgt_x, tgt_edge_index, tgt_edge_attr, W_self, W_nbr,
                 w_edge8, bias)
    return (h_src, h_tgt)


# speedup vs baseline: 1.6159x; 1.0153x over previous
import functools

import jax
import jax.numpy as jnp
from jax import lax
from jax.experimental import pallas as pl
from jax.experimental.pallas import tpu as pltpu
from jax.experimental.pallas import tpu_sc as plsc

NC = 2
NS = 16
NW = NC * NS
BATCH = 128
CHB = 512


def _sc_gather_scatter(x, src, dst, n_pad, k):
    n, d = x.shape
    rows_per_tile = n_pad // NS
    zeros_a = jnp.zeros((rows_per_tile, d), jnp.float32)

    mesh = plsc.VectorSubcoreMesh(
        core_axis_name="c", subcore_axis_name="s", num_cores=NC,
        num_subcores=NS)

    @functools.partial(
        pl.kernel,
        out_type=jax.ShapeDtypeStruct((NC, n_pad, d), jnp.float32),
        mesh=mesh,
        scratch_types=[
            pltpu.VMEM_SHARED((n_pad, d), jnp.float32),
            pltpu.VMEM((2, BATCH), jnp.int32),
            pltpu.VMEM((2, BATCH), jnp.int32),
            pltpu.VMEM((2, BATCH, d), jnp.float32),
            pltpu.SemaphoreType.DMA,
            pltpu.SemaphoreType.DMA,
        ],
    )
    def k_fn(x_hbm, src_hbm, dst_hbm, za_hbm, out_a,
             a_sh, isrc, idst, rows, gsem, ssem):
        cid = lax.axis_index("c")
        sid = lax.axis_index("s")
        wid = cid * NS + sid

        row0 = sid * rows_per_tile
        pltpu.sync_copy(za_hbm, a_sh.at[pl.ds(row0, rows_per_tile)])
        plsc.subcore_barrier()

        def half(g, b, first):
            if not first:
                pltpu.make_async_copy(
                    rows.at[b], a_sh.at[idst.at[b]], ssem).wait()
            base = (wid * k + g) * BATCH
            pltpu.sync_copy(src_hbm.at[pl.ds(base, BATCH)], isrc.at[b])
            pltpu.sync_copy(dst_hbm.at[pl.ds(base, BATCH)], idst.at[b])
            pltpu.async_copy(x_hbm.at[isrc.at[b]], rows.at[b], gsem).wait()
            pltpu.async_copy(rows.at[b], a_sh.at[idst.at[b]], ssem,
                             add=True)

        half(0, 0, True)
        half(1, 1, True)

        def body(i, _):
            g = i * 2
            half(g, 0, False)
            half(g + 1, 1, False)
            return _

        lax.fori_loop(1, k // 2, body, None)
        pltpu.make_async_copy(rows.at[0], a_sh.at[idst.at[0]], ssem).wait()
        pltpu.make_async_copy(rows.at[1], a_sh.at[idst.at[1]], ssem).wait()

        plsc.subcore_barrier()

        pltpu.sync_copy(a_sh.at[pl.ds(row0, rows_per_tile)],
                        out_a.at[cid, pl.ds(row0, rows_per_tile)])

    return k_fn(x, src, dst, zeros_a)


def _sc_edge_attr_sums(dst, ea16, n_pad, k2):
    nb = n_pad * 8 + 8
    zeros_b = jnp.zeros((nb,), jnp.float32)

    mesh = plsc.VectorSubcoreMesh(
        core_axis_name="c", subcore_axis_name="s", num_cores=NC,
        num_subcores=NS)

    @functools.partial(
        pl.kernel,
        out_type=jax.ShapeDtypeStruct((NW, n_pad * 8), jnp.float32),
        mesh=mesh,
        scratch_types=[
            pltpu.VMEM((nb,), jnp.float32),
            pltpu.VMEM((CHB,), jnp.int32),
            pltpu.VMEM((CHB * 16,), jnp.float32),
        ],
    )
    def k_fn(dst_hbm, ea_hbm, zb_hbm, out_b, bacc, dv, eav):
        cid = lax.axis_index("c")
        sid = lax.axis_index("s")
        wid = cid * NS + sid

        pltpu.sync_copy(zb_hbm, bacc)

        def chunk(g, _):
            base = (wid * k2 + g) * CHB
            pltpu.sync_copy(dst_hbm.at[pl.ds(base, CHB)], dv)
            pltpu.sync_copy(ea_hbm.at[pl.ds(base * 16, CHB * 16)], eav)

            def group(q, _):
                dv16 = dv[pl.ds(q * 16, 16)] * 8
                for j in range(16):
                    row = dv16[j]
                    vals = eav[pl.ds((q * 16 + j) * 16, 16)]
                    plsc.addupdate(bacc.at[pl.ds(row, 16)], vals)
                return _

            lax.fori_loop(0, CHB // 16, group, None)
            return _

        lax.fori_loop(0, k2, chunk, None)
        pltpu.sync_copy(bacc.at[pl.ds(0, n_pad * 8)], out_b.at[wid])

    return k_fn(dst, ea16, zeros_b)


def _tc_combine(x, a_part, b_part, w_self, w_nbr, w_edge8, bias):
    n, d = x.shape
    bn = 1000
    grid = (n // bn,)

    def body(x_ref, a_ref, b_ref, ws_ref, wn_ref, we_ref, bias_ref, o_ref):
        agg = a_ref[0] + a_ref[1]
        bsum = jnp.sum(b_ref[...], axis=0)
        wn = wn_ref[...]
        w2 = jnp.dot(we_ref[...], wn, preferred_element_type=jnp.float32)
        acc = jnp.dot(x_ref[...], ws_ref[...],
                      preferred_element_type=jnp.float32)
        acc += jnp.dot(agg, wn, preferred_element_type=jnp.float32)
        acc += jnp.dot(bsum, w2, preferred_element_type=jnp.float32)
        acc += bias_ref[...]
        o_ref[...] = jnp.maximum(acc, 0.0)

    return pl.pallas_call(
        body,
        grid=grid,
        in_specs=[
            pl.BlockSpec((bn, d), lambda i: (i, 0)),
            pl.BlockSpec((NC, bn, d), lambda i: (0, i, 0)),
            pl.BlockSpec((NW, bn, 8), lambda i: (0, i, 0)),
            pl.BlockSpec((d, d), lambda i: (0, 0)),
            pl.BlockSpec((d, d), lambda i: (0, 0)),
            pl.BlockSpec((8, d), lambda i: (0, 0)),
            pl.BlockSpec((1, d), lambda i: (0, 0)),
        ],
        out_specs=pl.BlockSpec((bn, d), lambda i: (i, 0)),
        out_shape=jax.ShapeDtypeStruct((n, d), jnp.float32),
    )(x, a_part, b_part, w_self, w_nbr, w_edge8, bias)


def _gnn(x, edge_index, edge_attr, w_self, w_nbr, w_edge8, bias):
    n, d = x.shape
    e = edge_index.shape[1]
    n_pad = (n + 1 + NS * 8 - 1) // (NS * 8) * (NS * 8)
    step = NW * max(BATCH, CHB)
    e_pad = -(-e // step) * step
    k = e_pad // (NW * BATCH)
    k2 = e_pad // (NW * CHB)

    src = jnp.pad(edge_index[0].astype(jnp.int32), (0, e_pad - e))
    dst = jnp.pad(edge_index[1].astype(jnp.int32), (0, e_pad - e),
                  constant_values=n)
    ea16 = jnp.pad(edge_attr.astype(jnp.float32),
                   ((0, e_pad - e), (0, 16 - edge_attr.shape[1])))

    a_part = _sc_gather_scatter(x, src, dst, n_pad, k)
    b_flat = _sc_edge_attr_sums(dst, ea16.reshape(-1), n_pad, k2)
    b_part = b_flat.reshape(NW, n_pad, 8)
    return _tc_combine(x, a_part, b_part, w_self, w_nbr, w_edge8, bias)


def kernel(source_batch, target_batch, src_x, src_edge_index, src_edge_attr,
           tgt_x, tgt_edge_index, tgt_edge_attr, W_self, W_nbr, W_edge, b):
    w_edge8 = jnp.pad(W_edge.astype(jnp.float32), ((0, 8 - W_edge.shape[0]),
                                                   (0, 0)))
    bias = b.astype(jnp.float32).reshape(1, -1)
    h_src = _gnn(src_x, src_edge_index, src_edge_attr, W_self, W_nbr,
                 w_edge8, bias)
    h_tgt = _gnn(tgt_x, tgt_edge_index, tgt_edge_attr, W_self, W_nbr,
                 w_edge8, bias)
    return (h_src, h_tgt)


# speedup vs baseline: 1.7385x; 1.0759x over previous
import functools

import jax
import jax.numpy as jnp
from jax import lax
from jax.experimental import pallas as pl
from jax.experimental.pallas import tpu as pltpu
from jax.experimental.pallas import tpu_sc as plsc

NC = 2
NS = 16
NW = NC * NS
BATCH = 128
CHB = 512


def _sc_gather_scatter(x, src, dst, n_pad, k):
    n, d = x.shape
    rows_per_tile = n_pad // NS
    zeros_a = jnp.zeros((rows_per_tile, d), jnp.float32)

    mesh = plsc.VectorSubcoreMesh(
        core_axis_name="c", subcore_axis_name="s", num_cores=NC,
        num_subcores=NS)

    ib = 16
    nblk = k // ib

    @functools.partial(
        pl.kernel,
        out_type=jax.ShapeDtypeStruct((NC, n_pad, d), jnp.float32),
        mesh=mesh,
        scratch_types=[
            pltpu.VMEM_SHARED((n_pad, d), jnp.float32),
            pltpu.VMEM((2, ib, BATCH), jnp.int32),
            pltpu.VMEM((2, ib, BATCH), jnp.int32),
            pltpu.VMEM((2, BATCH, d), jnp.float32),
            pltpu.SemaphoreType.DMA,
            pltpu.SemaphoreType.DMA,
            pltpu.SemaphoreType.DMA,
        ],
    )
    def k_fn(x_hbm, src_hbm, dst_hbm, za_hbm, out_a,
             a_sh, isrc, idst, rows, gsem, ssem, isem):
        cid = lax.axis_index("c")
        sid = lax.axis_index("s")
        wid = cid * NS + sid

        row0 = sid * rows_per_tile
        pltpu.sync_copy(za_hbm, a_sh.at[pl.ds(row0, rows_per_tile)])
        plsc.subcore_barrier()

        def load_idx(j, jb):
            r = wid * k + j * ib
            pltpu.async_copy(src_hbm.at[pl.ds(r, ib)], isrc.at[jb], isem)
            pltpu.async_copy(dst_hbm.at[pl.ds(r, ib)], idst.at[jb], isem)

        def wait_idx():
            pltpu.make_async_copy(src_hbm.at[pl.ds(0, ib)], isrc.at[0],
                                  isem).wait()
            pltpu.make_async_copy(dst_hbm.at[pl.ds(0, ib)], idst.at[0],
                                  isem).wait()

        def step(jb, t, first):
            b = t % 2
            if not first:
                pltpu.make_async_copy(
                    rows.at[b], a_sh.at[idst.at[jb, t]], ssem).wait()
            pltpu.async_copy(x_hbm.at[isrc.at[jb, t]], rows.at[b],
                             gsem).wait()
            pltpu.async_copy(rows.at[b], a_sh.at[idst.at[jb, t]], ssem,
                             add=True)

        def run_block(j, jb, prefetch):
            wait_idx()
            for t in range(ib):
                step(jb, t, False)
                if t == 1 and prefetch:
                    @pl.when(j + 1 < nblk)
                    def _pf():
                        load_idx(j + 1, 1 - jb)

        load_idx(0, 0)
        wait_idx()
        load_idx(1, 1)
        for t in range(ib):
            step(0, t, first=(t < 2))

        def pair(i, _):
            run_block(2 * i + 1, 1, True)
            run_block(2 * i + 2, 0, True)
            return _

        lax.fori_loop(0, (nblk - 1) // 2, pair, None)
        if (nblk - 1) % 2 == 1:
            run_block(nblk - 1, (nblk - 1) % 2, False)
        pltpu.make_async_copy(rows.at[0], a_sh.at[idst.at[0, 0]],
                              ssem).wait()
        pltpu.make_async_copy(rows.at[1], a_sh.at[idst.at[0, 1]],
                              ssem).wait()

        plsc.subcore_barrier()

        pltpu.sync_copy(a_sh.at[pl.ds(row0, rows_per_tile)],
                        out_a.at[cid, pl.ds(row0, rows_per_tile)])

    return k_fn(x, src, dst, zeros_a)


def _sc_edge_attr_sums(dst, ea16, n_pad, k2):
    nb = n_pad * 8 + 8
    zeros_b = jnp.zeros((nb,), jnp.float32)

    mesh = plsc.VectorSubcoreMesh(
        core_axis_name="c", subcore_axis_name="s", num_cores=NC,
        num_subcores=NS)

    @functools.partial(
        pl.kernel,
        out_type=jax.ShapeDtypeStruct((NW, n_pad * 8), jnp.float32),
        mesh=mesh,
        scratch_types=[
            pltpu.VMEM((nb,), jnp.float32),
            pltpu.VMEM((CHB,), jnp.int32),
            pltpu.VMEM((CHB * 16,), jnp.float32),
        ],
    )
    def k_fn(dst_hbm, ea_hbm, zb_hbm, out_b, bacc, dv, eav):
        cid = lax.axis_index("c")
        sid = lax.axis_index("s")
        wid = cid * NS + sid

        pltpu.sync_copy(zb_hbm, bacc)

        def chunk(g, _):
            base = (wid * k2 + g) * CHB
            pltpu.sync_copy(dst_hbm.at[pl.ds(base, CHB)], dv)
            pltpu.sync_copy(ea_hbm.at[pl.ds(base * 16, CHB * 16)], eav)

            def group(q, _):
                dv16 = dv[pl.ds(q * 16, 16)] * 8
                for j in range(16):
                    row = dv16[j]
                    vals = eav[pl.ds((q * 16 + j) * 16, 16)]
                    plsc.addupdate(bacc.at[pl.ds(row, 16)], vals)
                return _

            lax.fori_loop(0, CHB // 16, group, None)
            return _

        lax.fori_loop(0, k2, chunk, None)
        pltpu.sync_copy(bacc.at[pl.ds(0, n_pad * 8)], out_b.at[wid])

    return k_fn(dst, ea16, zeros_b)


def _tc_combine(x, a_part, b_part, w_self, w_nbr, w_edge8, bias):
    n, d = x.shape
    bn = 1000
    grid = (n // bn,)

    def body(x_ref, a_ref, b_ref, ws_ref, wn_ref, we_ref, bias_ref, o_ref):
        agg = a_ref[0] + a_ref[1]
        bsum = jnp.sum(b_ref[...], axis=0)
        wn = wn_ref[...]
        w2 = jnp.dot(we_ref[...], wn, preferred_element_type=jnp.float32)
        acc = jnp.dot(x_ref[...], ws_ref[...],
                      preferred_element_type=jnp.float32)
        acc += jnp.dot(agg, wn, preferred_element_type=jnp.float32)
        acc += jnp.dot(bsum, w2, preferred_element_type=jnp.float32)
        acc += bias_ref[...]
        o_ref[...] = jnp.maximum(acc, 0.0)

    return pl.pallas_call(
        body,
        grid=grid,
        in_specs=[
            pl.BlockSpec((bn, d), lambda i: (i, 0)),
            pl.BlockSpec((NC, bn, d), lambda i: (0, i, 0)),
            pl.BlockSpec((NW, bn, 8), lambda i: (0, i, 0)),
            pl.BlockSpec((d, d), lambda i: (0, 0)),
            pl.BlockSpec((d, d), lambda i: (0, 0)),
            pl.BlockSpec((8, d), lambda i: (0, 0)),
            pl.BlockSpec((1, d), lambda i: (0, 0)),
        ],
        out_specs=pl.BlockSpec((bn, d), lambda i: (i, 0)),
        out_shape=jax.ShapeDtypeStruct((n, d), jnp.float32),
    )(x, a_part, b_part, w_self, w_nbr, w_edge8, bias)


def _gnn(x, edge_index, edge_attr, w_self, w_nbr, w_edge8, bias):
    n, d = x.shape
    e = edge_index.shape[1]
    n_pad = (n + 1 + NS * 8 - 1) // (NS * 8) * (NS * 8)
    step = NW * max(BATCH, CHB)
    e_pad = -(-e // step) * step
    k = e_pad // (NW * BATCH)
    k2 = e_pad // (NW * CHB)

    src = jnp.pad(edge_index[0].astype(jnp.int32), (0, e_pad - e))
    dst = jnp.pad(edge_index[1].astype(jnp.int32), (0, e_pad - e),
                  constant_values=n)
    ea16 = jnp.pad(edge_attr.astype(jnp.float32),
                   ((0, e_pad - e), (0, 16 - edge_attr.shape[1])))

    a_part = _sc_gather_scatter(x, src.reshape(-1, BATCH),
                                dst.reshape(-1, BATCH), n_pad, k)
    b_flat = _sc_edge_attr_sums(dst, ea16.reshape(-1), n_pad, k2)
    b_part = b_flat.reshape(NW, n_pad, 8)
    return _tc_combine(x, a_part, b_part, w_self, w_nbr, w_edge8, bias)


def kernel(source_batch, target_batch, src_x, src_edge_index, src_edge_attr,
           tgt_x, tgt_edge_index, tgt_edge_attr, W_self, W_nbr, W_edge, b):
    w_edge8 = jnp.pad(W_edge.astype(jnp.float32), ((0, 8 - W_edge.shape[0]),
                                                   (0, 0)))
    bias = b.astype(jnp.float32).reshape(1, -1)
    h_src = _gnn(src_x, src_edge_index, src_edge_attr, W_self, W_nbr,
                 w_edge8, bias)
    h_tgt = _gnn(tgt_x, tgt_edge_index, tgt_edge_attr, W_self, W_nbr,
                 w_edge8, bias)
    return (h_src, h_tgt)


# speedup vs baseline: 1.8123x; 1.0425x over previous
import functools

import jax
import jax.numpy as jnp
from jax import lax
from jax.experimental import pallas as pl
from jax.experimental.pallas import tpu as pltpu
from jax.experimental.pallas import tpu_sc as plsc

NC = 2
NS = 16
NW = NC * NS
BATCH = 128
CHB = 512


def _sc_gather_scatter(x, src, dst, n_pad, k):
    n, d = x.shape
    rows_per_tile = n_pad // NS
    zeros_a = jnp.zeros((rows_per_tile, d), jnp.float32)

    mesh = plsc.VectorSubcoreMesh(
        core_axis_name="c", subcore_axis_name="s", num_cores=NC,
        num_subcores=NS)

    ib = 16
    nblk = k // ib

    @functools.partial(
        pl.kernel,
        out_type=jax.ShapeDtypeStruct((NC, n_pad, d), jnp.float32),
        mesh=mesh,
        scratch_types=[
            pltpu.VMEM_SHARED((n_pad, d), jnp.float32),
            pltpu.VMEM((2, ib, BATCH), jnp.int32),
            pltpu.VMEM((2, ib, BATCH), jnp.int32),
            pltpu.VMEM((2, BATCH, d), jnp.float32),
            pltpu.SemaphoreType.DMA,
            pltpu.SemaphoreType.DMA,
            pltpu.SemaphoreType.DMA,
            pltpu.SemaphoreType.DMA,
        ],
    )
    def k_fn(x_hbm, src_hbm, dst_hbm, za_hbm, out_a,
             a_sh, isrc, idst, rows, gsem0, gsem1, ssem, isem):
        cid = lax.axis_index("c")
        sid = lax.axis_index("s")
        wid = cid * NS + sid

        row0 = sid * rows_per_tile
        pltpu.sync_copy(za_hbm, a_sh.at[pl.ds(row0, rows_per_tile)])
        plsc.subcore_barrier()

        def load_idx(j, jb):
            r = wid * k + j * ib
            pltpu.async_copy(src_hbm.at[pl.ds(r, ib)], isrc.at[jb], isem)
            pltpu.async_copy(dst_hbm.at[pl.ds(r, ib)], idst.at[jb], isem)

        def wait_idx():
            pltpu.make_async_copy(src_hbm.at[pl.ds(0, ib)], isrc.at[0],
                                  isem).wait()
            pltpu.make_async_copy(dst_hbm.at[pl.ds(0, ib)], idst.at[0],
                                  isem).wait()

        gsems = (gsem0, gsem1)

        def run_block(j, jb, prefetch):
            wait_idx()
            pltpu.async_copy(x_hbm.at[isrc.at[jb, 0]], rows.at[0], gsem0)
            pltpu.async_copy(x_hbm.at[isrc.at[jb, 1]], rows.at[1], gsem1)
            for t in range(ib):
                b = t % 2
                pltpu.make_async_copy(x_hbm.at[isrc.at[jb, t]], rows.at[b],
                                      gsems[b]).wait()
                pltpu.async_copy(rows.at[b], a_sh.at[idst.at[jb, t]], ssem,
                                 add=True)
                pltpu.make_async_copy(rows.at[b], a_sh.at[idst.at[jb, t]],
                                      ssem).wait()
                if t + 2 < ib:
                    pltpu.async_copy(x_hbm.at[isrc.at[jb, t + 2]],
                                     rows.at[b], gsems[b])
                if t == 1 and prefetch:
                    @pl.when(j + 1 < nblk)
                    def _pf():
                        load_idx(j + 1, 1 - jb)

        load_idx(0, 0)
        run_block(0, 0, True)

        def pair(i, _):
            run_block(2 * i + 1, 1, True)
            run_block(2 * i + 2, 0, True)
            return _

        lax.fori_loop(0, (nblk - 1) // 2, pair, None)
        if (nblk - 1) % 2 == 1:
            run_block(nblk - 1, (nblk - 1) % 2, False)

        plsc.subcore_barrier()

        pltpu.sync_copy(a_sh.at[pl.ds(row0, rows_per_tile)],
                        out_a.at[cid, pl.ds(row0, rows_per_tile)])

    return k_fn(x, src, dst, zeros_a)


def _sc_edge_attr_sums(dst, ea16, n_pad, k2):
    nb = n_pad * 8 + 8
    zeros_b = jnp.zeros((nb,), jnp.float32)

    mesh = plsc.VectorSubcoreMesh(
        core_axis_name="c", subcore_axis_name="s", num_cores=NC,
        num_subcores=NS)

    @functools.partial(
        pl.kernel,
        out_type=jax.ShapeDtypeStruct((NW, n_pad * 8), jnp.float32),
        mesh=mesh,
        scratch_types=[
            pltpu.VMEM((nb,), jnp.float32),
            pltpu.VMEM((CHB,), jnp.int32),
            pltpu.VMEM((CHB * 16,), jnp.float32),
        ],
    )
    def k_fn(dst_hbm, ea_hbm, zb_hbm, out_b, bacc, dv, eav):
        cid = lax.axis_index("c")
        sid = lax.axis_index("s")
        wid = cid * NS + sid

        pltpu.sync_copy(zb_hbm, bacc)

        def chunk(g, _):
            base = (wid * k2 + g) * CHB
            pltpu.sync_copy(dst_hbm.at[pl.ds(base, CHB)], dv)
            pltpu.sync_copy(ea_hbm.at[pl.ds(base * 16, CHB * 16)], eav)

            def group(q, _):
                dv16 = dv[pl.ds(q * 16, 16)] * 8
                for j in range(16):
                    row = dv16[j]
                    vals = eav[pl.ds((q * 16 + j) * 16, 16)]
                    plsc.addupdate(bacc.at[pl.ds(row, 16)], vals)
                return _

            lax.fori_loop(0, CHB // 16, group, None)
            return _

        lax.fori_loop(0, k2, chunk, None)
        pltpu.sync_copy(bacc.at[pl.ds(0, n_pad * 8)], out_b.at[wid])

    return k_fn(dst, ea16, zeros_b)


def _tc_combine(x, a_part, b_part, w_self, w_nbr, w_edge8, bias):
    n, d = x.shape
    bn = 1000
    grid = (n // bn,)

    def body(x_ref, a_ref, b_ref, ws_ref, wn_ref, we_ref, bias_ref, o_ref):
        agg = a_ref[0] + a_ref[1]
        bsum = jnp.sum(b_ref[...], axis=0)
        wn = wn_ref[...]
        w2 = jnp.dot(we_ref[...], wn, preferred_element_type=jnp.float32)
        acc = jnp.dot(x_ref[...], ws_ref[...],
                      preferred_element_type=jnp.float32)
        acc += jnp.dot(agg, wn, preferred_element_type=jnp.float32)
        acc += jnp.dot(bsum, w2, preferred_element_type=jnp.float32)
        acc += bias_ref[...]
        o_ref[...] = jnp.maximum(acc, 0.0)

    return pl.pallas_call(
        body,
        grid=grid,
        in_specs=[
            pl.BlockSpec((bn, d), lambda i: (i, 0)),
            pl.BlockSpec((NC, bn, d), lambda i: (0, i, 0)),
            pl.BlockSpec((NW, bn, 8), lambda i: (0, i, 0)),
            pl.BlockSpec((d, d), lambda i: (0, 0)),
            pl.BlockSpec((d, d), lambda i: (0, 0)),
            pl.BlockSpec((8, d), lambda i: (0, 0)),
            pl.BlockSpec((1, d), lambda i: (0, 0)),
        ],
        out_specs=pl.BlockSpec((bn, d), lambda i: (i, 0)),
        out_shape=jax.ShapeDtypeStruct((n, d), jnp.float32),
    )(x, a_part, b_part, w_self, w_nbr, w_edge8, bias)


def _gnn(x, edge_index, edge_attr, w_self, w_nbr, w_edge8, bias):
    n, d = x.shape
    e = edge_index.shape[1]
    n_pad = (n + 1 + NS * 8 - 1) // (NS * 8) * (NS * 8)
    step = NW * max(BATCH, CHB)
    e_pad = -(-e // step) * step
    k = e_pad // (NW * BATCH)
    k2 = e_pad // (NW * CHB)

    src = jnp.pad(edge_index[0].astype(jnp.int32), (0, e_pad - e))
    dst = jnp.pad(edge_index[1].astype(jnp.int32), (0, e_pad - e),
                  constant_values=n)
    ea16 = jnp.pad(edge_attr.astype(jnp.float32),
                   ((0, e_pad - e), (0, 16 - edge_attr.shape[1])))

    a_part = _sc_gather_scatter(x, src.reshape(-1, BATCH),
                                dst.reshape(-1, BATCH), n_pad, k)
    b_flat = _sc_edge_attr_sums(dst, ea16.reshape(-1), n_pad, k2)
    b_part = b_flat.reshape(NW, n_pad, 8)
    return _tc_combine(x, a_part, b_part, w_self, w_nbr, w_edge8, bias)


def kernel(source_batch, target_batch, src_x, src_edge_index, src_edge_attr,
           tgt_x, tgt_edge_index, tgt_edge_attr, W_self, W_nbr, W_edge, b):
    w_edge8 = jnp.pad(W_edge.astype(jnp.float32), ((0, 8 - W_edge.shape[0]),
                                                   (0, 0)))
    bias = b.astype(jnp.float32).reshape(1, -1)
    h_src = _gnn(src_x, src_edge_index, src_edge_attr, W_self, W_nbr,
                 w_edge8, bias)
    h_tgt = _gnn(tgt_x, tgt_edge_index, tgt_edge_attr, W_self, W_nbr,
                 w_edge8, bias)
    return (h_src, h_tgt)


# speedup vs baseline: 2.1324x; 1.1766x over previous
import functools

import jax
import jax.numpy as jnp
from jax import lax
from jax.experimental import pallas as pl
from jax.experimental.pallas import tpu as pltpu
from jax.experimental.pallas import tpu_sc as plsc

NC = 2
NS = 16
BATCH = 128
CHB = 512


def _sc_gather_scatter(x1, src1, dst1, x2, src2, dst2, n_pad, k):
    n, d = x1.shape
    rows_per_tile = n_pad // NS
    zeros_a = jnp.zeros((rows_per_tile, d), jnp.float32)

    mesh = plsc.VectorSubcoreMesh(
        core_axis_name="c", subcore_axis_name="s", num_cores=NC,
        num_subcores=NS)

    ib = 16
    nblk = k // ib

    @functools.partial(
        pl.kernel,
        out_type=jax.ShapeDtypeStruct((NC, n_pad, d), jnp.float32),
        mesh=mesh,
        scratch_types=[
            pltpu.VMEM_SHARED((n_pad, d), jnp.float32),
            pltpu.VMEM((2, ib, BATCH), jnp.int32),
            pltpu.VMEM((2, ib, BATCH), jnp.int32),
            pltpu.VMEM((2, BATCH, d), jnp.float32),
            pltpu.SemaphoreType.DMA,
            pltpu.SemaphoreType.DMA,
            pltpu.SemaphoreType.DMA,
            pltpu.SemaphoreType.DMA,
        ],
    )
    def k_fn(x1_hbm, src1_hbm, dst1_hbm, x2_hbm, src2_hbm, dst2_hbm, za_hbm,
             out_a, a_sh, isrc, idst, rows, gsem0, gsem1, ssem, isem):
        cid = lax.axis_index("c")
        sid = lax.axis_index("s")

        row0 = sid * rows_per_tile
        pltpu.sync_copy(za_hbm, a_sh.at[pl.ds(row0, rows_per_tile)])
        plsc.subcore_barrier()

        gsems = (gsem0, gsem1)

        def pipe(x_hbm, src_hbm, dst_hbm):
            def load_idx(j, jb):
                r = sid * k + j * ib
                pltpu.async_copy(src_hbm.at[pl.ds(r, ib)], isrc.at[jb],
                                 isem)
                pltpu.async_copy(dst_hbm.at[pl.ds(r, ib)], idst.at[jb],
                                 isem)

            def wait_idx():
                pltpu.make_async_copy(src_hbm.at[pl.ds(0, ib)], isrc.at[0],
                                      isem).wait()
                pltpu.make_async_copy(dst_hbm.at[pl.ds(0, ib)], idst.at[0],
                                      isem).wait()

            def run_block(j, jb, prefetch):
                wait_idx()
                pltpu.async_copy(x_hbm.at[isrc.at[jb, 0]], rows.at[0],
                                 gsem0)
                pltpu.async_copy(x_hbm.at[isrc.at[jb, 1]], rows.at[1],
                                 gsem1)
                for t in range(ib):
                    b = t % 2
                    pltpu.make_async_copy(x_hbm.at[isrc.at[jb, t]],
                                          rows.at[b], gsems[b]).wait()
                    pltpu.async_copy(rows.at[b], a_sh.at[idst.at[jb, t]],
                                     ssem, add=True)
                    pltpu.make_async_copy(rows.at[b],
                                          a_sh.at[idst.at[jb, t]],
                                          ssem).wait()
                    if t + 2 < ib:
                        pltpu.async_copy(x_hbm.at[isrc.at[jb, t + 2]],
                                         rows.at[b], gsems[b])
                    if t == 1 and prefetch:
                        @pl.when(j + 1 < nblk)
                        def _pf():
                            load_idx(j + 1, 1 - jb)

            load_idx(0, 0)
            run_block(0, 0, True)

            def pair(i, _):
                run_block(2 * i + 1, 1, True)
                run_block(2 * i + 2, 0, True)
                return _

            lax.fori_loop(0, (nblk - 1) // 2, pair, None)
            if (nblk - 1) % 2 == 1:
                run_block(nblk - 1, (nblk - 1) % 2, False)

        @pl.when(cid == 0)
        def _graph1():
            pipe(x1_hbm, src1_hbm, dst1_hbm)

        @pl.when(cid == 1)
        def _graph2():
            pipe(x2_hbm, src2_hbm, dst2_hbm)

        plsc.subcore_barrier()

        pltpu.sync_copy(a_sh.at[pl.ds(row0, rows_per_tile)],
                        out_a.at[cid, pl.ds(row0, rows_per_tile)])

    return k_fn(x1, src1, dst1, x2, src2, dst2, zeros_a)


def _sc_edge_attr_sums(dst1, ea1, dst2, ea2, n_pad, k2):
    nb = n_pad * 8 + 8
    zeros_b = jnp.zeros((nb,), jnp.float32)

    mesh = plsc.VectorSubcoreMesh(
        core_axis_name="c", subcore_axis_name="s", num_cores=NC,
        num_subcores=NS)

    @functools.partial(
        pl.kernel,
        out_type=jax.ShapeDtypeStruct((NC * NS, n_pad * 8), jnp.float32),
        mesh=mesh,
        scratch_types=[
            pltpu.VMEM((nb,), jnp.float32),
            pltpu.VMEM((CHB,), jnp.int32),
            pltpu.VMEM((CHB * 16,), jnp.float32),
        ],
    )
    def k_fn(dst1_hbm, ea1_hbm, dst2_hbm, ea2_hbm, zb_hbm, out_b,
             bacc, dv, eav):
        cid = lax.axis_index("c")
        sid = lax.axis_index("s")
        wid = cid * NS + sid

        pltpu.sync_copy(zb_hbm, bacc)

        def pipe(dst_hbm, ea_hbm):
            def chunk(g, _):
                base = (sid * k2 + g) * CHB
                pltpu.sync_copy(dst_hbm.at[pl.ds(base, CHB)], dv)
                pltpu.sync_copy(ea_hbm.at[pl.ds(base * 16, CHB * 16)], eav)

                def group(q, _):
                    dv16 = dv[pl.ds(q * 16, 16)] * 8
                    for j in range(16):
                        row = dv16[j]
                        vals = eav[pl.ds((q * 16 + j) * 16, 16)]
                        plsc.addupdate(bacc.at[pl.ds(row, 16)], vals)
                    return _

                lax.fori_loop(0, CHB // 16, group, None)
                return _

            lax.fori_loop(0, k2, chunk, None)

        @pl.when(cid == 0)
        def _graph1():
            pipe(dst1_hbm, ea1_hbm)

        @pl.when(cid == 1)
        def _graph2():
            pipe(dst2_hbm, ea2_hbm)

        pltpu.sync_copy(bacc.at[pl.ds(0, n_pad * 8)], out_b.at[wid])

    return k_fn(dst1, ea1, dst2, ea2, zeros_b)


def _tc_combine(x, a_sum, b_part, w_self, w_nbr, w_edge8, bias):
    n, d = x.shape
    bn = 1000
    grid = (n // bn,)

    def body(x_ref, a_ref, b_ref, ws_ref, wn_ref, we_ref, bias_ref, o_ref):
        bsum = jnp.sum(b_ref[...], axis=0)
        wn = wn_ref[...]
        w2 = jnp.dot(we_ref[...], wn, preferred_element_type=jnp.float32)
        acc = jnp.dot(x_ref[...], ws_ref[...],
                      preferred_element_type=jnp.float32)
        acc += jnp.dot(a_ref[...], wn, preferred_element_type=jnp.float32)
        acc += jnp.dot(bsum, w2, preferred_element_type=jnp.float32)
        acc += bias_ref[...]
        o_ref[...] = jnp.maximum(acc, 0.0)

    return pl.pallas_call(
        body,
        grid=grid,
        in_specs=[
            pl.BlockSpec((bn, d), lambda i: (i, 0)),
            pl.BlockSpec((bn, d), lambda i: (i, 0)),
            pl.BlockSpec((NS, bn, 8), lambda i: (0, i, 0)),
            pl.BlockSpec((d, d), lambda i: (0, 0)),
            pl.BlockSpec((d, d), lambda i: (0, 0)),
            pl.BlockSpec((8, d), lambda i: (0, 0)),
            pl.BlockSpec((1, d), lambda i: (0, 0)),
        ],
        out_specs=pl.BlockSpec((bn, d), lambda i: (i, 0)),
        out_shape=jax.ShapeDtypeStruct((n, d), jnp.float32),
    )(x, a_sum, b_part, w_self, w_nbr, w_edge8, bias)


def _pad_edges(edge_index, edge_attr, e_pad, n):
    e = edge_index.shape[1]
    src = jnp.pad(edge_index[0].astype(jnp.int32), (0, e_pad - e))
    dst = jnp.pad(edge_index[1].astype(jnp.int32), (0, e_pad - e),
                  constant_values=n)
    ea16 = jnp.pad(edge_attr.astype(jnp.float32),
                   ((0, e_pad - e), (0, 16 - edge_attr.shape[1])))
    return src, dst, ea16


def kernel(source_batch, target_batch, src_x, src_edge_index, src_edge_attr,
           tgt_x, tgt_edge_index, tgt_edge_attr, W_self, W_nbr, W_edge, b):
    w_edge8 = jnp.pad(W_edge.astype(jnp.float32), ((0, 8 - W_edge.shape[0]),
                                                   (0, 0)))
    bias = b.astype(jnp.float32).reshape(1, -1)

    n, d = src_x.shape
    n_pad = (n + 1 + NS * 8 - 1) // (NS * 8) * (NS * 8)
    step = NS * max(BATCH, CHB)
    e_max = max(src_edge_index.shape[1], tgt_edge_index.shape[1])
    e_pad = -(-e_max // step) * step
    k = e_pad // (NS * BATCH)
    k2 = e_pad // (NS * CHB)

    src1, dst1, ea1 = _pad_edges(src_edge_index, src_edge_attr, e_pad, n)
    src2, dst2, ea2 = _pad_edges(tgt_edge_index, tgt_edge_attr, e_pad, n)

    a_sum = _sc_gather_scatter(
        src_x, src1.reshape(-1, BATCH), dst1.reshape(-1, BATCH),
        tgt_x, src2.reshape(-1, BATCH), dst2.reshape(-1, BATCH), n_pad, k)
    b_flat = _sc_edge_attr_sums(dst1, ea1.reshape(-1), dst2, ea2.reshape(-1),
                                n_pad, k2)
    b_part = b_flat.reshape(NC, NS, n_pad, 8)

    h_src = _tc_combine(src_x, a_sum[0], b_part[0], W_self, W_nbr,
                        w_edge8, bias)
    h_tgt = _tc_combine(tgt_x, a_sum[1], b_part[1], W_self, W_nbr,
                        w_edge8, bias)
    return (h_src, h_tgt)
